# Initial kernel scaffold; baseline (speedup 1.0000x reference)
import jax
import jax.numpy as jnp
from jax.experimental import pallas as pl


def kernel(context, word_emb, pos_emb):
    raise NotImplementedError("write your pallas kernel here")


# speedup vs baseline: 1.3186x; 1.3186x over previous
import functools
import math

import jax
import jax.numpy as jnp
from jax import lax
from jax.experimental import pallas as pl
from jax.experimental.pallas import tpu as pltpu
from jax.experimental.pallas import tpu_sc as plsc

DIM = 128
LANES = 16
SCALE = math.sqrt(float(DIM))
NUM_CORES = 2
NUM_SUBCORES = 16
NW = NUM_CORES * NUM_SUBCORES
IDX_CHUNK = 128


def _emb_body(n_total, seq_len, ctx_hbm, we_hbm, pos_hbm, out_hbm,
              idx_v, rows_v, pos_v, sem):
    rows_per_w = n_total // NW
    n_chunks = rows_per_w // IDX_CHUNK
    wid = lax.axis_index("s") * NUM_CORES + lax.axis_index("c")
    base = wid * rows_per_w

    pltpu.sync_copy(ctx_hbm.at[pl.ds(wid * n_chunks, n_chunks)], idx_v)

    copies = []
    for j in range(n_chunks):
        copies.append(
            pltpu.async_copy(
                we_hbm.at[idx_v.at[j]],
                rows_v.at[pl.ds(j * IDX_CHUNK, IDX_CHUNK)],
                sem,
            )
        )
    pbase = lax.rem(base, seq_len)
    pltpu.sync_copy(pos_hbm.at[pl.ds(pbase, rows_per_w)], pos_v)
    for cp in copies:
        cp.wait()

    def row_body(r, carry):
        for j in range(DIM // LANES):
            sl = pl.ds(j * LANES, LANES)
            rows_v[r, sl] = rows_v[r, sl] * SCALE + pos_v[r, sl]
        return carry

    lax.fori_loop(0, rows_per_w, row_body, 0)

    pltpu.sync_copy(rows_v, out_hbm.at[pl.ds(base, rows_per_w)])


def kernel(context, word_emb, pos_emb):
    b, l = context.shape
    n_total = b * l
    rows_per_w = n_total // NW
    ctx = context.reshape(n_total // IDX_CHUNK, IDX_CHUNK).astype(jnp.int32)

    mesh = plsc.VectorSubcoreMesh(core_axis_name="c", subcore_axis_name="s")
    body = functools.partial(_emb_body, n_total, l)
    out = pl.kernel(
        body,
        mesh=mesh,
        out_type=jax.ShapeDtypeStruct((n_total, DIM), jnp.float32),
        scratch_types=[
            pltpu.VMEM((rows_per_w // IDX_CHUNK, IDX_CHUNK), jnp.int32),
            pltpu.VMEM((rows_per_w, DIM), jnp.float32),
            pltpu.VMEM((rows_per_w, DIM), jnp.float32),
            pltpu.SemaphoreType.DMA,
        ],
    )(ctx, word_emb, pos_emb)
    return out.reshape(b, l, DIM)
